# baseline (speedup 1.0000x reference)
import functools

import jax
import jax.numpy as jnp
from jax import lax
from jax.experimental import pallas as pl
from jax.experimental.pallas import tpu as pltpu
from jax.experimental.pallas import tpu_sc as plsc

B, T, N, E = 32, 64, 80, 2560
OUT, HID = 8, 24
G = B * T
NC, NS, L = 2, 16, 16
NW = NC * NS
GPW = G // NW
NCHUNK = E // L
NXC = N // L


def _gat_sc(xf, wf, edge_index, wn8, al8, ar8, we8, ae8):
    mesh = plsc.VectorSubcoreMesh(core_axis_name="c", subcore_axis_name="s")

    @functools.partial(
        pl.kernel, mesh=mesh,
        out_type=jax.ShapeDtypeStruct((G,), jnp.float32),
        scratch_types=[
            pltpu.VMEM((2, E), jnp.int32),
            pltpu.VMEM((GPW * N,), jnp.float32),
            pltpu.VMEM((2 * E,), jnp.float32),
            pltpu.VMEM((N,), jnp.float32),
            pltpu.VMEM((N,), jnp.float32),
            pltpu.VMEM((GPW,), jnp.float32),
            pltpu.VMEM((5 * L,), jnp.float32),
            pltpu.VMEM((N,), jnp.float32),
            pltpu.VMEM((N,), jnp.float32),
            pltpu.SemaphoreType.DMA,
            pltpu.SemaphoreType.DMA,
        ],
        compiler_params=pltpu.CompilerParams(needs_layout_passes=False),
    )
    def k(x_hbm, w_hbm, ei_hbm, wn_hbm, al_hbm, ar_hbm, we_hbm, ae_hbm,
          m_hbm, eiv, xblk, wbuf, denom, num, mout, wsc, xg, xrg, sem0, sem1):
        wid = lax.axis_index("s") * NC + lax.axis_index("c")
        base = wid * GPW
        pltpu.sync_copy(ei_hbm, eiv)
        for j, ref in enumerate((wn_hbm, al_hbm, ar_hbm, we_hbm, ae_hbm)):
            pltpu.sync_copy(ref, wsc.at[pl.ds(j * L, OUT)])
        pltpu.sync_copy(x_hbm.at[pl.ds(base * N, GPW * N)], xblk)
        zero16 = jnp.zeros((L,), jnp.float32)
        izero = lax.iota(jnp.int32, L) * 0
        dn = lax.GatherDimensionNumbers(offset_dims=(), collapsed_slice_dims=(0,),
                                        start_index_map=(0,))

        def bfly_sum(v):
            for sh in (8, 4, 2, 1):
                idx = lax.iota(jnp.int32, L) ^ sh
                v = v + lax.gather(v, idx[:, None], dn, slice_sizes=(1,),
                                   mode=lax.GatherScatterMode.PROMISE_IN_BOUNDS)
            return v

        lane8 = lax.iota(jnp.int32, L) < OUT
        wn = wsc[pl.ds(0, L)]
        al = wsc[pl.ds(L, L)]
        ar = wsc[pl.ds(2 * L, L)]
        we = wsc[pl.ds(3 * L, L)]
        ae = wsc[pl.ds(4 * L, L)]
        cl = bfly_sum(jnp.where(lane8, wn * al, 0.0))
        cr = bfly_sum(jnp.where(lane8, wn * ar, 0.0))
        ce = bfly_sum(jnp.where(lane8, we * ae, 0.0))
        kcoef = jnp.abs(cl) + jnp.abs(cr)
        kbias = jnp.abs(ce)

        def process_graph(gi, wb):
            for cj in range(NXC):
                denom[pl.ds(cj * L, L)] = zero16
                num[pl.ds(cj * L, L)] = zero16
            gbase = gi * N
            amax = zero16
            for cj in range(NXC):
                sl = pl.ds(cj * L, L)
                v = xblk[pl.ds(gbase + cj * L, L)]
                xg[sl] = v
                xrg[sl] = cr * v
                amax = jnp.maximum(amax, jnp.abs(v))
            for sh in (8, 4, 2, 1):
                idx = lax.iota(jnp.int32, L) ^ sh
                amax = jnp.maximum(amax, lax.gather(
                    amax, idx[:, None], dn, slice_sizes=(1,),
                    mode=lax.GatherScatterMode.PROMISE_IN_BOUNDS))
            K = kcoef * amax + kbias

            @plsc.parallel_loop(0, NCHUNK, 1, unroll=16)
            def chunk_body(ci):
                sl = pl.ds(ci * L, L)
                si = eiv[0, sl]
                di = eiv[1, sl]
                xs = plsc.load_gather(xg, [si])
                xdr = plsc.load_gather(xrg, [di])
                wv = wbuf[pl.ds(wb * E + ci * L, L)]
                e = cl * xs + xdr + ce * wv
                e = jnp.maximum(e, 0.2 * e)
                p = jnp.exp(e - K)
                plsc.addupdate_scatter(denom, [di], p)
                plsc.addupdate_scatter(num, [di], p * xs)

            s = zero16
            for cj in range(NXC):
                sl = pl.ds(cj * L, L)
                s = s + num[sl] / (denom[sl] + 1e-9)
            for sh in (8, 4, 2, 1):
                idx = lax.iota(jnp.int32, L) ^ sh
                s = s + lax.gather(s, idx[:, None], dn, slice_sizes=(1,),
                                   mode=lax.GatherScatterMode.PROMISE_IN_BOUNDS)
            plsc.store_scatter(mout, [izero + gi], s * (1.0 / N))

        def wcopy(g, b, sem):
            return pltpu.make_async_copy(w_hbm.at[pl.ds(g * E, E)],
                                         wbuf.at[pl.ds(b * E, E)], sem)

        wcopy(base, 0, sem0).start()

        def pair_body(gp, _):
            g0 = 2 * gp
            wcopy(base + g0, 0, sem0).wait()
            wcopy(base + g0 + 1, 1, sem1).start()
            process_graph(g0, 0)
            wcopy(base + g0 + 1, 1, sem1).wait()

            @pl.when(gp + 1 < GPW // 2)
            def _prefetch():
                wcopy(base + g0 + 2, 0, sem0).start()

            process_graph(g0 + 1, 1)
            return _

        lax.fori_loop(0, GPW // 2, pair_body, 0)
        pltpu.sync_copy(mout, m_hbm.at[pl.ds(base, GPW)])

    return k(xf, wf, edge_index, wn8, al8, ar8, we8, ae8)


def _lstm_tc(m_tb1, wnode, wih, whh, gbias, bih, bhh, fcw, fcb, fccw, fccb):
    def body(m_ref, wn_ref, wih_ref, whh_ref, gb_ref, bih_ref, bhh_ref,
             fcw_ref, fcb_ref, fccw_ref, fccb_ref, out_ref):
        def dotT(a, b):
            return lax.dot_general(a, b, (((1,), (1,)), ((), ())),
                                   preferred_element_type=jnp.float32)

        def dot(a, b):
            return lax.dot_general(a, b, (((1,), (0,)), ((), ())),
                                   preferred_element_type=jnp.float32)

        wn = wn_ref[...]
        wih = wih_ref[...]
        whh = whh_ref[...]
        vbase = dotT(gb_ref[...], wih) + bih_ref[...] + bhh_ref[...]
        vfull = dotT(wn, wih)
        vi, vf, vg, vo = [vfull[:, j * HID:(j + 1) * HID] for j in range(4)]
        bi, bf, bg, bo = [vbase[:, j * HID:(j + 1) * HID] for j in range(4)]
        whhT = jnp.transpose(whh)
        wi, wf, wg, wo = [whhT[:, j * HID:(j + 1) * HID] for j in range(4)]
        m2 = lax.dot_general(fcw_ref[...], fccw_ref[...],
                             (((0,), (1,)), ((), ())),
                             preferred_element_type=jnp.float32)
        b2v = dotT(fcb_ref[...], fccw_ref[...]) + fccb_ref[...]

        def step(t, carry):
            h, c = carry
            op = dot(h, m2) + b2v

            @pl.when(t > 0)
            def _():
                out_ref[:, t - 1, :] = op

            mt = m_ref[t]
            i = jax.nn.sigmoid(mt * vi + bi + dot(h, wi))
            f = jax.nn.sigmoid(mt * vf + bf + dot(h, wf))
            g = jnp.tanh(mt * vg + bg + dot(h, wg))
            o = jax.nn.sigmoid(mt * vo + bo + dot(h, wo))
            c = f * c + i * g
            h = o * jnp.tanh(c)
            return (h, c)

        h0 = jnp.zeros((B, HID), jnp.float32)
        c0 = jnp.zeros((B, HID), jnp.float32)
        hT, _cT = lax.fori_loop(0, T, step, (h0, c0))
        out_ref[:, T - 1, :] = dot(hT, m2) + b2v

    return pl.pallas_call(
        body,
        out_shape=jax.ShapeDtypeStruct((B, T, 20), jnp.float32),
    )(m_tb1, wnode, wih, whh, gbias, bih, bhh, fcw, fcb, fccw, fccb)


def kernel(x, edge_index, edge_weight, W_node, a_l, a_r, W_edge, a_e,
           gat_bias, W_ih, W_hh, b_ih, b_hh, fc_W, fc_b, fcc_W, fcc_b):
    xf = x.reshape(G * N)
    wf = edge_weight.reshape(G * E)

    m = _gat_sc(xf, wf, edge_index, W_node.reshape(OUT), a_l, a_r,
                W_edge.reshape(OUT), a_e)

    m_tb1 = m.reshape(B, T).T.reshape(T, B, 1)
    out = _lstm_tc(m_tb1, W_node, W_ih, W_hh, gat_bias.reshape(1, OUT),
                   b_ih.reshape(1, 4 * HID), b_hh.reshape(1, 4 * HID),
                   fc_W, fc_b.reshape(1, -1), fcc_W,
                   fcc_b.reshape(1, -1))
    return out.reshape(G, 20)


# speedup vs baseline: 1.0432x; 1.0432x over previous
import functools

import jax
import jax.numpy as jnp
from jax import lax
from jax.experimental import pallas as pl
from jax.experimental.pallas import tpu as pltpu
from jax.experimental.pallas import tpu_sc as plsc

B, T, N, E = 32, 64, 80, 2560
OUT, HID = 8, 24
G = B * T
NC, NS, L = 2, 16, 16
NW = NC * NS
GPW = G // NW
NCHUNK = E // L
NXC = N // L


def _gat_sc(xf, wf, edge_index, wn8, al8, ar8, we8, ae8):
    mesh = plsc.VectorSubcoreMesh(core_axis_name="c", subcore_axis_name="s")

    @functools.partial(
        pl.kernel, mesh=mesh,
        out_type=jax.ShapeDtypeStruct((G,), jnp.float32),
        scratch_types=[
            pltpu.VMEM((2, E), jnp.int32),
            pltpu.VMEM((GPW * N,), jnp.float32),
            pltpu.VMEM((2 * E,), jnp.float32),
            pltpu.VMEM((N,), jnp.float32),
            pltpu.VMEM((N,), jnp.float32),
            pltpu.VMEM((GPW,), jnp.float32),
            pltpu.VMEM((5 * L,), jnp.float32),
            pltpu.VMEM((N,), jnp.float32),
            pltpu.VMEM((N,), jnp.float32),
            pltpu.SemaphoreType.DMA,
            pltpu.SemaphoreType.DMA,
        ],
        compiler_params=pltpu.CompilerParams(needs_layout_passes=False),
    )
    def k(x_hbm, w_hbm, ei_hbm, wn_hbm, al_hbm, ar_hbm, we_hbm, ae_hbm,
          m_hbm, eiv, xblk, wbuf, denom, num, mout, wsc, xg, xrg, sem0, sem1):
        wid = lax.axis_index("s") * NC + lax.axis_index("c")
        base = wid * GPW
        pltpu.sync_copy(ei_hbm, eiv)
        for j, ref in enumerate((wn_hbm, al_hbm, ar_hbm, we_hbm, ae_hbm)):
            pltpu.sync_copy(ref, wsc.at[pl.ds(j * L, OUT)])
        pltpu.sync_copy(x_hbm.at[pl.ds(base * N, GPW * N)], xblk)
        zero16 = jnp.zeros((L,), jnp.float32)
        izero = lax.iota(jnp.int32, L) * 0
        dn = lax.GatherDimensionNumbers(offset_dims=(), collapsed_slice_dims=(0,),
                                        start_index_map=(0,))

        def bfly_sum(v):
            for sh in (8, 4, 2, 1):
                idx = lax.iota(jnp.int32, L) ^ sh
                v = v + lax.gather(v, idx[:, None], dn, slice_sizes=(1,),
                                   mode=lax.GatherScatterMode.PROMISE_IN_BOUNDS)
            return v

        lane8 = lax.iota(jnp.int32, L) < OUT
        wn = wsc[pl.ds(0, L)]
        al = wsc[pl.ds(L, L)]
        ar = wsc[pl.ds(2 * L, L)]
        we = wsc[pl.ds(3 * L, L)]
        ae = wsc[pl.ds(4 * L, L)]
        cl = bfly_sum(jnp.where(lane8, wn * al, 0.0))
        cr = bfly_sum(jnp.where(lane8, wn * ar, 0.0))
        ce = bfly_sum(jnp.where(lane8, we * ae, 0.0))
        kcoef = jnp.abs(cl) + jnp.abs(cr)
        kbias = jnp.abs(ce)

        def process_graph(gi, wb):
            for cj in range(NXC):
                denom[pl.ds(cj * L, L)] = zero16
                num[pl.ds(cj * L, L)] = zero16
            gbase = gi * N
            amax = zero16
            for cj in range(NXC):
                sl = pl.ds(cj * L, L)
                v = xblk[pl.ds(gbase + cj * L, L)]
                xg[sl] = v
                xrg[sl] = cr * v
                amax = jnp.maximum(amax, jnp.abs(v))
            for sh in (8, 4, 2, 1):
                idx = lax.iota(jnp.int32, L) ^ sh
                amax = jnp.maximum(amax, lax.gather(
                    amax, idx[:, None], dn, slice_sizes=(1,),
                    mode=lax.GatherScatterMode.PROMISE_IN_BOUNDS))
            K = kcoef * amax + kbias

            @plsc.parallel_loop(0, NCHUNK, 1, unroll=8)
            def chunk_body(ci):
                sl = pl.ds(ci * L, L)
                si = eiv[0, sl]
                di = eiv[1, sl]
                xs = plsc.load_gather(xg, [si])
                xdr = plsc.load_gather(xrg, [di])
                wv = wbuf[pl.ds(wb * E + ci * L, L)]
                e = cl * xs + xdr + ce * wv
                e = jnp.maximum(e, 0.2 * e)
                p = jnp.exp(e - K)
                plsc.addupdate_scatter(denom, [di], p)
                plsc.addupdate_scatter(num, [di], p * xs)

            s = zero16
            for cj in range(NXC):
                sl = pl.ds(cj * L, L)
                s = s + num[sl] / (denom[sl] + 1e-9)
            for sh in (8, 4, 2, 1):
                idx = lax.iota(jnp.int32, L) ^ sh
                s = s + lax.gather(s, idx[:, None], dn, slice_sizes=(1,),
                                   mode=lax.GatherScatterMode.PROMISE_IN_BOUNDS)
            plsc.store_scatter(mout, [izero + gi], s * (1.0 / N))

        def wcopy(g, b, sem):
            return pltpu.make_async_copy(w_hbm.at[pl.ds(g * E, E)],
                                         wbuf.at[pl.ds(b * E, E)], sem)

        wcopy(base, 0, sem0).start()

        def pair_body(gp, _):
            g0 = 2 * gp
            wcopy(base + g0, 0, sem0).wait()
            wcopy(base + g0 + 1, 1, sem1).start()
            process_graph(g0, 0)
            wcopy(base + g0 + 1, 1, sem1).wait()

            @pl.when(gp + 1 < GPW // 2)
            def _prefetch():
                wcopy(base + g0 + 2, 0, sem0).start()

            process_graph(g0 + 1, 1)
            return _

        lax.fori_loop(0, GPW // 2, pair_body, 0)
        pltpu.sync_copy(mout, m_hbm.at[pl.ds(base, GPW)])

    return k(xf, wf, edge_index, wn8, al8, ar8, we8, ae8)


def _lstm_tc(m_tb1, wnode, wih, whh, gbias, bih, bhh, fcw, fcb, fccw, fccb):
    def body(m_ref, wn_ref, wih_ref, whh_ref, gb_ref, bih_ref, bhh_ref,
             fcw_ref, fcb_ref, fccw_ref, fccb_ref, out_ref):
        def dotT(a, b):
            return lax.dot_general(a, b, (((1,), (1,)), ((), ())),
                                   preferred_element_type=jnp.float32)

        def dot(a, b):
            return lax.dot_general(a, b, (((1,), (0,)), ((), ())),
                                   preferred_element_type=jnp.float32)

        wn = wn_ref[...]
        wih = wih_ref[...]
        whh = whh_ref[...]
        vbase = dotT(gb_ref[...], wih) + bih_ref[...] + bhh_ref[...]
        vfull = dotT(wn, wih)
        vi, vf, vg, vo = [vfull[:, j * HID:(j + 1) * HID] for j in range(4)]
        bi, bf, bg, bo = [vbase[:, j * HID:(j + 1) * HID] for j in range(4)]
        whhT = jnp.transpose(whh)
        wi, wf, wg, wo = [whhT[:, j * HID:(j + 1) * HID] for j in range(4)]
        m2 = lax.dot_general(fcw_ref[...], fccw_ref[...],
                             (((0,), (1,)), ((), ())),
                             preferred_element_type=jnp.float32)
        b2v = dotT(fcb_ref[...], fccw_ref[...]) + fccb_ref[...]

        def step(t, carry):
            h, c = carry
            op = dot(h, m2) + b2v

            @pl.when(t > 0)
            def _():
                out_ref[:, t - 1, :] = op

            mt = m_ref[t]
            i = jax.nn.sigmoid(mt * vi + bi + dot(h, wi))
            f = jax.nn.sigmoid(mt * vf + bf + dot(h, wf))
            g = jnp.tanh(mt * vg + bg + dot(h, wg))
            o = jax.nn.sigmoid(mt * vo + bo + dot(h, wo))
            c = f * c + i * g
            h = o * jnp.tanh(c)
            return (h, c)

        h0 = jnp.zeros((B, HID), jnp.float32)
        c0 = jnp.zeros((B, HID), jnp.float32)
        hT, _cT = lax.fori_loop(0, T, step, (h0, c0))
        out_ref[:, T - 1, :] = dot(hT, m2) + b2v

    return pl.pallas_call(
        body,
        out_shape=jax.ShapeDtypeStruct((B, T, 20), jnp.float32),
    )(m_tb1, wnode, wih, whh, gbias, bih, bhh, fcw, fcb, fccw, fccb)


def kernel(x, edge_index, edge_weight, W_node, a_l, a_r, W_edge, a_e,
           gat_bias, W_ih, W_hh, b_ih, b_hh, fc_W, fc_b, fcc_W, fcc_b):
    xf = x.reshape(G * N)
    wf = edge_weight.reshape(G * E)

    m = _gat_sc(xf, wf, edge_index, W_node.reshape(OUT), a_l, a_r,
                W_edge.reshape(OUT), a_e)

    m_tb1 = m.reshape(B, T).T.reshape(T, B, 1)
    out = _lstm_tc(m_tb1, W_node, W_ih, W_hh, gat_bias.reshape(1, OUT),
                   b_ih.reshape(1, 4 * HID), b_hh.reshape(1, 4 * HID),
                   fc_W, fc_b.reshape(1, -1), fcc_W,
                   fcc_b.reshape(1, -1))
    return out.reshape(G, 20)


# speedup vs baseline: 1.0891x; 1.0439x over previous
import functools

import jax
import jax.numpy as jnp
from jax import lax
from jax.experimental import pallas as pl
from jax.experimental.pallas import tpu as pltpu
from jax.experimental.pallas import tpu_sc as plsc

B, T, N, E = 32, 64, 80, 2560
OUT, HID = 8, 24
G = B * T
NC, NS, L = 2, 16, 16
NW = NC * NS
GPW = G // NW
NCHUNK = E // L
NXC = N // L


def _gat_sc(xf, wf, edge_index, wn8, al8, ar8, we8, ae8):
    mesh = plsc.VectorSubcoreMesh(core_axis_name="c", subcore_axis_name="s")

    @functools.partial(
        pl.kernel, mesh=mesh,
        out_type=jax.ShapeDtypeStruct((G,), jnp.float32),
        scratch_types=[
            pltpu.VMEM((2, E), jnp.int32),
            pltpu.VMEM((GPW * N,), jnp.float32),
            pltpu.VMEM((2 * E,), jnp.float32),
            pltpu.VMEM((N,), jnp.float32),
            pltpu.VMEM((N,), jnp.float32),
            pltpu.VMEM((GPW,), jnp.float32),
            pltpu.VMEM((5 * L,), jnp.float32),
            pltpu.VMEM((N,), jnp.float32),
            pltpu.VMEM((N,), jnp.float32),
            pltpu.VMEM((E,), jnp.int32),
            pltpu.SemaphoreType.DMA,
            pltpu.SemaphoreType.DMA,
        ],
        compiler_params=pltpu.CompilerParams(needs_layout_passes=False),
    )
    def k(x_hbm, w_hbm, ei_hbm, wn_hbm, al_hbm, ar_hbm, we_hbm, ae_hbm,
          m_hbm, eiv, xblk, wbuf, denom, num, mout, wsc, xg, xrg, piv,
          sem0, sem1):
        wid = lax.axis_index("s") * NC + lax.axis_index("c")
        base = wid * GPW
        pltpu.sync_copy(ei_hbm, eiv)
        for j, ref in enumerate((wn_hbm, al_hbm, ar_hbm, we_hbm, ae_hbm)):
            pltpu.sync_copy(ref, wsc.at[pl.ds(j * L, OUT)])
        pltpu.sync_copy(x_hbm.at[pl.ds(base * N, GPW * N)], xblk)
        zero16 = jnp.zeros((L,), jnp.float32)
        izero = lax.iota(jnp.int32, L) * 0
        dn = lax.GatherDimensionNumbers(offset_dims=(), collapsed_slice_dims=(0,),
                                        start_index_map=(0,))

        def bfly_sum(v):
            for sh in (8, 4, 2, 1):
                idx = lax.iota(jnp.int32, L) ^ sh
                v = v + lax.gather(v, idx[:, None], dn, slice_sizes=(1,),
                                   mode=lax.GatherScatterMode.PROMISE_IN_BOUNDS)
            return v

        lane8 = lax.iota(jnp.int32, L) < OUT
        wn = wsc[pl.ds(0, L)]
        al = wsc[pl.ds(L, L)]
        ar = wsc[pl.ds(2 * L, L)]
        we = wsc[pl.ds(3 * L, L)]
        ae = wsc[pl.ds(4 * L, L)]
        cl = bfly_sum(jnp.where(lane8, wn * al, 0.0))
        cr = bfly_sum(jnp.where(lane8, wn * ar, 0.0))
        ce = bfly_sum(jnp.where(lane8, we * ae, 0.0))
        kcoef = jnp.abs(cl) + jnp.abs(cr)
        kbias = jnp.abs(ce)

        @plsc.parallel_loop(0, NCHUNK, 1, unroll=4)
        def pack_body(ci):
            sl = pl.ds(ci * L, L)
            piv[sl] = (eiv[1, sl] << 8) | eiv[0, sl]

        def process_graph(gi, wb):
            for cj in range(NXC):
                denom[pl.ds(cj * L, L)] = zero16
                num[pl.ds(cj * L, L)] = zero16
            gbase = gi * N
            amax = zero16
            for cj in range(NXC):
                sl = pl.ds(cj * L, L)
                v = xblk[pl.ds(gbase + cj * L, L)]
                xg[sl] = v
                xrg[sl] = cr * v
                amax = jnp.maximum(amax, jnp.abs(v))
            for sh in (8, 4, 2, 1):
                idx = lax.iota(jnp.int32, L) ^ sh
                amax = jnp.maximum(amax, lax.gather(
                    amax, idx[:, None], dn, slice_sizes=(1,),
                    mode=lax.GatherScatterMode.PROMISE_IN_BOUNDS))
            K = kcoef * amax + kbias

            @plsc.parallel_loop(0, NCHUNK, 1, unroll=8)
            def chunk_body(ci):
                sl = pl.ds(ci * L, L)
                pk = piv[sl]
                si = pk & 255
                di = pk >> 8
                xs = plsc.load_gather(xg, [si])
                xdr = plsc.load_gather(xrg, [di])
                wv = wbuf[pl.ds(wb * E + ci * L, L)]
                e = cl * xs + xdr + ce * wv
                e = jnp.maximum(e, 0.2 * e)
                p = jnp.exp(e - K)
                plsc.addupdate_scatter(denom, [di], p)
                plsc.addupdate_scatter(num, [di], p * xs)

            s = zero16
            for cj in range(NXC):
                sl = pl.ds(cj * L, L)
                s = s + num[sl] / (denom[sl] + 1e-9)
            for sh in (8, 4, 2, 1):
                idx = lax.iota(jnp.int32, L) ^ sh
                s = s + lax.gather(s, idx[:, None], dn, slice_sizes=(1,),
                                   mode=lax.GatherScatterMode.PROMISE_IN_BOUNDS)
            plsc.store_scatter(mout, [izero + gi], s * (1.0 / N))

        def wcopy(g, b, sem):
            return pltpu.make_async_copy(w_hbm.at[pl.ds(g * E, E)],
                                         wbuf.at[pl.ds(b * E, E)], sem)

        wcopy(base, 0, sem0).start()

        def pair_body(gp, _):
            g0 = 2 * gp
            wcopy(base + g0, 0, sem0).wait()
            wcopy(base + g0 + 1, 1, sem1).start()
            process_graph(g0, 0)
            wcopy(base + g0 + 1, 1, sem1).wait()

            @pl.when(gp + 1 < GPW // 2)
            def _prefetch():
                wcopy(base + g0 + 2, 0, sem0).start()

            process_graph(g0 + 1, 1)
            return _

        lax.fori_loop(0, GPW // 2, pair_body, 0)
        pltpu.sync_copy(mout, m_hbm.at[pl.ds(base, GPW)])

    return k(xf, wf, edge_index, wn8, al8, ar8, we8, ae8)


def _lstm_tc(m_tb1, wnode, wih, whh, gbias, bih, bhh, fcw, fcb, fccw, fccb):
    def body(m_ref, wn_ref, wih_ref, whh_ref, gb_ref, bih_ref, bhh_ref,
             fcw_ref, fcb_ref, fccw_ref, fccb_ref, out_ref):
        def dotT(a, b):
            return lax.dot_general(a, b, (((1,), (1,)), ((), ())),
                                   preferred_element_type=jnp.float32)

        def dot(a, b):
            return lax.dot_general(a, b, (((1,), (0,)), ((), ())),
                                   preferred_element_type=jnp.float32)

        wn = wn_ref[...]
        wih = wih_ref[...]
        whh = whh_ref[...]
        vbase = dotT(gb_ref[...], wih) + bih_ref[...] + bhh_ref[...]
        vfull = dotT(wn, wih)
        vi, vf, vg, vo = [vfull[:, j * HID:(j + 1) * HID] for j in range(4)]
        bi, bf, bg, bo = [vbase[:, j * HID:(j + 1) * HID] for j in range(4)]
        whhT = jnp.transpose(whh)
        wi, wf, wg, wo = [whhT[:, j * HID:(j + 1) * HID] for j in range(4)]
        m2 = lax.dot_general(fcw_ref[...], fccw_ref[...],
                             (((0,), (1,)), ((), ())),
                             preferred_element_type=jnp.float32)
        b2v = dotT(fcb_ref[...], fccw_ref[...]) + fccb_ref[...]

        def step(t, carry):
            h, c = carry
            op = dot(h, m2) + b2v

            @pl.when(t > 0)
            def _():
                out_ref[:, t - 1, :] = op

            mt = m_ref[t]
            i = jax.nn.sigmoid(mt * vi + bi + dot(h, wi))
            f = jax.nn.sigmoid(mt * vf + bf + dot(h, wf))
            g = jnp.tanh(mt * vg + bg + dot(h, wg))
            o = jax.nn.sigmoid(mt * vo + bo + dot(h, wo))
            c = f * c + i * g
            h = o * jnp.tanh(c)
            return (h, c)

        h0 = jnp.zeros((B, HID), jnp.float32)
        c0 = jnp.zeros((B, HID), jnp.float32)
        hT, _cT = lax.fori_loop(0, T, step, (h0, c0))
        out_ref[:, T - 1, :] = dot(hT, m2) + b2v

    return pl.pallas_call(
        body,
        out_shape=jax.ShapeDtypeStruct((B, T, 20), jnp.float32),
    )(m_tb1, wnode, wih, whh, gbias, bih, bhh, fcw, fcb, fccw, fccb)


def kernel(x, edge_index, edge_weight, W_node, a_l, a_r, W_edge, a_e,
           gat_bias, W_ih, W_hh, b_ih, b_hh, fc_W, fc_b, fcc_W, fcc_b):
    xf = x.reshape(G * N)
    wf = edge_weight.reshape(G * E)

    m = _gat_sc(xf, wf, edge_index, W_node.reshape(OUT), a_l, a_r,
                W_edge.reshape(OUT), a_e)

    m_tb1 = m.reshape(B, T).T.reshape(T, B, 1)
    out = _lstm_tc(m_tb1, W_node, W_ih, W_hh, gat_bias.reshape(1, OUT),
                   b_ih.reshape(1, 4 * HID), b_hh.reshape(1, 4 * HID),
                   fc_W, fc_b.reshape(1, -1), fcc_W,
                   fcc_b.reshape(1, -1))
    return out.reshape(G, 20)
